# baseline (speedup 1.0000x reference)
import jax
import jax.numpy as jnp
from jax.experimental import pallas as pl
from jax.experimental.pallas import tpu as pltpu

_INPUT = 768
_HIDDEN = 8192
_K = 128
_TB = 128
_NCHAIN = 2
_SLAB = _HIDDEN // 16

_INT_MIN = -2147483648


def _key_bits(k):
    return jnp.where(k < 0, k ^ 0x7FFFFFFF, k)


def _stage(xc, wt_ref, w_ref, b1_ref, b2_ref, o_ref, oacc_ref,
           pre_enc, prebf_enc, pre_sel, prebf_sel, msk_sel, msk_dec):
    rows = _TB // _NCHAIN
    chains = [pl.ds(c * rows, rows) for c in range(_NCHAIN)]

    def p1_body(i, ps):
        cols = pl.ds(i * _SLAB, _SLAB)
        val = (
            jnp.dot(xc, wt_ref[:, cols], preferred_element_type=jnp.float32)
            + b1_ref[:, cols]
        )
        pre_enc[:, cols] = val
        prebf_enc[:, cols] = val.astype(jnp.bfloat16)

        bit = jnp.left_shift(jnp.int32(1), 15 - i)
        out = []
        for c in range(_NCHAIN):
            p = ps[c]
            cu = p | bit
            t = cu ^ 0x8000
            t16 = t - jnp.where(t >= 32768, 65536, 0)
            bits16 = jnp.where(t16 < 0, t16 ^ 0x7FFF, t16)
            cand_f = jax.lax.bitcast_convert_type(
                jnp.left_shift(bits16, 16), jnp.float32)
            cand_bf = cand_f.astype(jnp.bfloat16)
            maskb = jnp.where(prebf_sel[chains[c], :] >= cand_bf,
                              jnp.bfloat16(1), jnp.bfloat16(0))
            w = _HIDDEN
            mb = maskb
            while w > 128:
                w //= 2
                mb = mb[:, :w] + mb[:, w:2 * w]
            cnt = jnp.sum(mb.astype(jnp.float32), axis=1, keepdims=True)
            out.append(jnp.where(cnt >= _K, cu, p))
        return tuple(out)

    p16 = jax.lax.fori_loop(
        0, 16, p1_body,
        tuple(jnp.zeros((rows, 1), jnp.int32) for _ in range(_NCHAIN)))

    los, his = [], []
    for c in range(_NCHAIN):
        t = p16[c] ^ 0x8000
        c16 = t - jnp.where(t >= 32768, 65536, 0)
        lo = jnp.clip(c16 * 65536, _INT_MIN + 32768, 2147483647 - 98303)
        lo = lo - 32768
        los.append(lo)
        his.append(lo + 131071)

    oacc_ref[...] = jnp.zeros((_TB, _INPUT), jnp.float32)

    def p2_body(i, state):
        @pl.when(i < 16)
        def _decode_slab():
            cols = pl.ds(i * _SLAB, _SLAB)
            oacc_ref[...] += jnp.dot(msk_dec[:, cols], w_ref[cols, :],
                                     preferred_element_type=jnp.float32)

        out = []
        for c in range(_NCHAIN):
            lo, hi = state[2 * c], state[2 * c + 1]
            mid = lo + jax.lax.shift_right_logical(hi - lo, 1)
            cand_f = jax.lax.bitcast_convert_type(_key_bits(mid), jnp.float32)
            m01 = jnp.where(pre_sel[chains[c], :] >= cand_f, 1.0, 0.0)
            cnt = jnp.sum(m01, axis=1, keepdims=True)
            ge = cnt >= _K
            out.append(jnp.where(ge, mid, lo))
            out.append(jnp.where(ge, hi, mid - 1))
        return tuple(out)

    state = jax.lax.fori_loop(
        0, 17, p2_body,
        tuple(x for c in range(_NCHAIN) for x in (los[c], his[c])))

    thresh = jnp.concatenate(
        [jax.lax.bitcast_convert_type(_key_bits(state[2 * c]), jnp.float32)
         for c in range(_NCHAIN)], axis=0)

    pre = pre_sel[...]
    msk_sel[...] = jnp.where(pre >= thresh, pre, 0.0).astype(jnp.bfloat16)
    o_ref[...] = oacc_ref[...] + b2_ref[...]


def _sae_block(x_ref, wt_ref, w_ref, b1_ref, b2_ref, o_ref,
               pre0, pre1, prebf0, prebf1, msk0, msk1, oacc_ref):
    s = pl.program_id(0)
    xc = x_ref[...] - b2_ref[...]

    @pl.when(jax.lax.rem(s, 2) == 0)
    def _even():
        _stage(xc, wt_ref, w_ref, b1_ref, b2_ref, o_ref, oacc_ref,
               pre0, prebf0, pre1, prebf1, msk1, msk0)

    @pl.when(jax.lax.rem(s, 2) == 1)
    def _odd():
        _stage(xc, wt_ref, w_ref, b1_ref, b2_ref, o_ref, oacc_ref,
               pre1, prebf1, pre0, prebf0, msk0, msk1)


def kernel(x, W, WT, b1, b2):
    tokens = x.shape[0]
    nb = tokens // _TB
    w_bf16 = W.astype(jnp.bfloat16)
    b1r = b1.reshape(1, _HIDDEN)
    b2r = b2.reshape(1, _INPUT)
    plane_f = pltpu.VMEM((_TB, _HIDDEN), jnp.float32)
    plane_b = pltpu.VMEM((_TB, _HIDDEN), jnp.bfloat16)
    return pl.pallas_call(
        _sae_block,
        grid=(nb + 2,),
        in_specs=[
            pl.BlockSpec((_TB, _INPUT), lambda i: (jnp.minimum(i, nb - 1), 0)),
            pl.BlockSpec((_INPUT, _HIDDEN), lambda i: (0, 0)),
            pl.BlockSpec((_HIDDEN, _INPUT), lambda i: (0, 0)),
            pl.BlockSpec((1, _HIDDEN), lambda i: (0, 0)),
            pl.BlockSpec((1, _INPUT), lambda i: (0, 0)),
        ],
        out_specs=pl.BlockSpec((_TB, _INPUT),
                               lambda i: (jnp.clip(i - 2, 0, nb - 1), 0)),
        out_shape=jax.ShapeDtypeStruct((tokens, _INPUT), jnp.float32),
        scratch_shapes=[plane_f, plane_f, plane_b, plane_b, plane_b, plane_b,
                        pltpu.VMEM((_TB, _INPUT), jnp.float32)],
        compiler_params=pltpu.CompilerParams(
            dimension_semantics=("arbitrary",),
        ),
    )(x, WT, w_bf16, b1r, b2r)


# speedup vs baseline: 1.2328x; 1.2328x over previous
import jax
import jax.numpy as jnp
from jax.experimental import pallas as pl
from jax.experimental.pallas import tpu as pltpu

_INPUT = 768
_HIDDEN = 8192
_K = 128
_TB = 128
_NCHAIN = 2

_INT_MIN = -2147483648


def _key_bits(k):
    return jnp.where(k < 0, k ^ 0x7FFFFFFF, k)


def _sae_block(x_ref, wt_ref, w_ref, b1_ref, b2_ref, o_ref):
    xc = x_ref[...] - b2_ref[...]
    pre = (
        jnp.dot(xc, wt_ref[...], preferred_element_type=jnp.float32)
        + b1_ref[...]
    )
    pre_bf = pre.astype(jnp.bfloat16)

    rows = _TB // _NCHAIN
    chains = [slice(c * rows, (c + 1) * rows) for c in range(_NCHAIN)]
    pre_c = [pre[s] for s in chains]
    pre_bf_c = [pre_bf[s] for s in chains]

    def count_ge(x_f32chain, cand_f):
        m01 = jnp.where(x_f32chain >= cand_f, 1.0, 0.0)
        return jnp.sum(m01, axis=1, keepdims=True)

    def p1_body(i, ps):
        bit = jnp.left_shift(jnp.int32(1), 15 - i)
        out = []
        for c in range(_NCHAIN):
            p = ps[c]
            cu = p | bit
            t = cu ^ 0x8000
            t16 = t - jnp.where(t >= 32768, 65536, 0)
            bits16 = jnp.where(t16 < 0, t16 ^ 0x7FFF, t16)
            cand_f = jax.lax.bitcast_convert_type(
                jnp.left_shift(bits16, 16), jnp.float32)
            cand_bf = cand_f.astype(jnp.bfloat16)
            maskb = jnp.where(pre_bf_c[c] >= cand_bf,
                              jnp.bfloat16(1), jnp.bfloat16(0))
            s = maskb
            w = _HIDDEN
            while w > 128:
                w //= 2
                s = s[:, :w] + s[:, w:2 * w]
            cnt = jnp.sum(s.astype(jnp.float32), axis=1, keepdims=True)
            out.append(jnp.where(cnt >= _K, cu, p))
        return tuple(out)

    p16 = jax.lax.fori_loop(
        0, 16, p1_body,
        tuple(jnp.zeros((rows, 1), jnp.int32) for _ in range(_NCHAIN)))

    los, his = [], []
    for c in range(_NCHAIN):
        t = p16[c] ^ 0x8000
        c16 = t - jnp.where(t >= 32768, 65536, 0)
        lo = jnp.clip(c16 * 65536, _INT_MIN + 32768, 2147483647 - 98303)
        lo = lo - 32768
        los.append(lo)
        his.append(lo + 131071)

    def p2_body(i, state):
        out = []
        for c in range(_NCHAIN):
            lo, hi = state[2 * c], state[2 * c + 1]
            mid = lo + jax.lax.shift_right_logical(hi - lo, 1)
            cand_f = jax.lax.bitcast_convert_type(_key_bits(mid), jnp.float32)
            cnt = count_ge(pre_c[c], cand_f)
            ge = cnt >= _K
            out.append(jnp.where(ge, mid, lo))
            out.append(jnp.where(ge, hi, mid - 1))
        return tuple(out)

    state = jax.lax.fori_loop(
        0, 17, p2_body,
        tuple(x for c in range(_NCHAIN) for x in (los[c], his[c])))

    thresh = jnp.concatenate(
        [jax.lax.bitcast_convert_type(_key_bits(state[2 * c]), jnp.float32)
         for c in range(_NCHAIN)], axis=0)

    masked = jnp.where(pre >= thresh, pre, 0.0).astype(jnp.bfloat16)
    out = jnp.dot(masked, w_ref[...], preferred_element_type=jnp.float32)
    o_ref[...] = out + b2_ref[...]


def kernel(x, W, WT, b1, b2):
    tokens = x.shape[0]
    w_bf16 = W.astype(jnp.bfloat16)
    b1r = b1.reshape(1, _HIDDEN)
    b2r = b2.reshape(1, _INPUT)
    return pl.pallas_call(
        _sae_block,
        grid=(tokens // _TB,),
        in_specs=[
            pl.BlockSpec((_TB, _INPUT), lambda i: (i, 0)),
            pl.BlockSpec((_INPUT, _HIDDEN), lambda i: (0, 0)),
            pl.BlockSpec((_HIDDEN, _INPUT), lambda i: (0, 0)),
            pl.BlockSpec((1, _HIDDEN), lambda i: (0, 0)),
            pl.BlockSpec((1, _INPUT), lambda i: (0, 0)),
        ],
        out_specs=pl.BlockSpec((_TB, _INPUT), lambda i: (i, 0)),
        out_shape=jax.ShapeDtypeStruct((tokens, _INPUT), jnp.float32),
        compiler_params=pltpu.CompilerParams(
            dimension_semantics=("arbitrary",),
        ),
    )(x, WT, w_bf16, b1r, b2r)


# speedup vs baseline: 1.3670x; 1.1088x over previous
import jax
import jax.numpy as jnp
from jax.experimental import pallas as pl
from jax.experimental.pallas import tpu as pltpu

_INPUT = 768
_HIDDEN = 8192
_K = 128
_TB = 256
_NCHAIN = 2

_INT_MIN = -2147483648


def _key_bits(k):
    return jnp.where(k < 0, k ^ 0x7FFFFFFF, k)


def _sae_block(x_ref, wt_ref, w_ref, b1_ref, b2_ref, o_ref):
    xc = x_ref[...] - b2_ref[...]
    pre = (
        jnp.dot(xc, wt_ref[...], preferred_element_type=jnp.float32)
        + b1_ref[...]
    )
    pre_bf = pre.astype(jnp.bfloat16)

    rows = _TB // _NCHAIN
    chains = [slice(c * rows, (c + 1) * rows) for c in range(_NCHAIN)]
    pre_c = [pre[s] for s in chains]
    pre_bf_c = [pre_bf[s] for s in chains]

    def count_ge(x_f32chain, cand_f):
        m01 = jnp.where(x_f32chain >= cand_f, 1.0, 0.0)
        return jnp.sum(m01, axis=1, keepdims=True)

    def p1_body(i, ps):
        bit = jnp.left_shift(jnp.int32(1), 15 - i)
        out = []
        for c in range(_NCHAIN):
            p = ps[c]
            cu = p | bit
            t = cu ^ 0x8000
            t16 = t - jnp.where(t >= 32768, 65536, 0)
            bits16 = jnp.where(t16 < 0, t16 ^ 0x7FFF, t16)
            cand_f = jax.lax.bitcast_convert_type(
                jnp.left_shift(bits16, 16), jnp.float32)
            cand_bf = cand_f.astype(jnp.bfloat16)
            maskb = jnp.where(pre_bf_c[c] >= cand_bf,
                              jnp.bfloat16(1), jnp.bfloat16(0))
            s = maskb
            w = _HIDDEN
            while w > 128:
                w //= 2
                s = s[:, :w] + s[:, w:2 * w]
            cnt = jnp.sum(s.astype(jnp.float32), axis=1, keepdims=True)
            out.append(jnp.where(cnt >= _K, cu, p))
        return tuple(out)

    p16 = jax.lax.fori_loop(
        0, 16, p1_body,
        tuple(jnp.zeros((rows, 1), jnp.int32) for _ in range(_NCHAIN)))

    los, his = [], []
    for c in range(_NCHAIN):
        t = p16[c] ^ 0x8000
        c16 = t - jnp.where(t >= 32768, 65536, 0)
        lo = jnp.clip(c16 * 65536, _INT_MIN + 32768, 2147483647 - 98303)
        lo = lo - 32768
        los.append(lo)
        his.append(lo + 131071)

    def p2_body(i, state):
        out = []
        for c in range(_NCHAIN):
            lo, hi = state[2 * c], state[2 * c + 1]
            mid = lo + jax.lax.shift_right_logical(hi - lo, 1)
            cand_f = jax.lax.bitcast_convert_type(_key_bits(mid), jnp.float32)
            cnt = count_ge(pre_c[c], cand_f)
            ge = cnt >= _K
            out.append(jnp.where(ge, mid, lo))
            out.append(jnp.where(ge, hi, mid - 1))
        return tuple(out)

    state = jax.lax.fori_loop(
        0, 17, p2_body,
        tuple(x for c in range(_NCHAIN) for x in (los[c], his[c])))

    thresh = jnp.concatenate(
        [jax.lax.bitcast_convert_type(_key_bits(state[2 * c]), jnp.float32)
         for c in range(_NCHAIN)], axis=0)

    masked = jnp.where(pre >= thresh, pre, 0.0).astype(jnp.bfloat16)
    out = jnp.dot(masked, w_ref[...], preferred_element_type=jnp.float32)
    o_ref[...] = out + b2_ref[...]


def kernel(x, W, WT, b1, b2):
    tokens = x.shape[0]
    w_bf16 = W.astype(jnp.bfloat16)
    b1r = b1.reshape(1, _HIDDEN)
    b2r = b2.reshape(1, _INPUT)
    return pl.pallas_call(
        _sae_block,
        grid=(tokens // _TB,),
        in_specs=[
            pl.BlockSpec((_TB, _INPUT), lambda i: (i, 0)),
            pl.BlockSpec((_INPUT, _HIDDEN), lambda i: (0, 0)),
            pl.BlockSpec((_HIDDEN, _INPUT), lambda i: (0, 0)),
            pl.BlockSpec((1, _HIDDEN), lambda i: (0, 0)),
            pl.BlockSpec((1, _INPUT), lambda i: (0, 0)),
        ],
        out_specs=pl.BlockSpec((_TB, _INPUT), lambda i: (i, 0)),
        out_shape=jax.ShapeDtypeStruct((tokens, _INPUT), jnp.float32),
        compiler_params=pltpu.CompilerParams(
            dimension_semantics=("arbitrary",),
        ),
    )(x, WT, w_bf16, b1r, b2r)
